# baseline (speedup 1.0000x reference)
import functools

import jax
import jax.numpy as jnp
from jax.experimental import pallas as pl

INPUT_DIM = 768
N_EXPERTS = 8
EXPERT_DIM = 128
HID = N_EXPERTS * EXPERT_DIM
TOP_K = 2
GRID = 2
NSX = 8
NSW = 4
TILE_B = 2048 // (GRID * NSX)
W1_ROWS = HID // NSW


def _gating(logits):
    ids = jax.lax.broadcasted_iota(jnp.int32, logits.shape, 1)
    m1 = jnp.max(logits, axis=-1, keepdims=True)
    i1 = jnp.min(jnp.where(logits == m1, ids, N_EXPERTS),
                 axis=-1, keepdims=True)
    masked = jnp.where(ids == i1, -jnp.inf, logits)
    m2 = jnp.max(masked, axis=-1, keepdims=True)
    i2 = jnp.min(jnp.where(masked == m2, ids, N_EXPERTS),
                 axis=-1, keepdims=True)
    e2 = jnp.exp(m2 - m1)
    denom = 1.0 + e2
    return (jnp.where(ids == i1, 1.0 / denom, 0.0) +
            jnp.where(ids == i2, e2 / denom, 0.0))


def _tile(xt, wg, w1s, b1, w2, b2):
    logits = jax.lax.dot_general(
        xt, wg, (((1,), (1,)), ((), ())),
        preferred_element_type=jnp.float32)
    gw = _gating(logits)

    hparts = [jax.lax.dot_general(
        xt, w1, (((1,), (1,)), ((), ())),
        preferred_element_type=jnp.float32) for w1 in w1s]
    h = (hparts[0] if len(hparts) == 1
         else jnp.concatenate(hparts, axis=1)) + b1
    h = 0.5 * h * (1.0 + jax.lax.erf(h * 0.7071067811865476))

    hs = jnp.concatenate(
        [h[:, e * EXPERT_DIM:(e + 1) * EXPERT_DIM] * gw[:, e:e + 1]
         for e in range(N_EXPERTS)], axis=1)
    out = jax.lax.dot_general(
        hs, w2, (((1,), (0,)), ((), ())),
        preferred_element_type=jnp.float32)
    out = out + jax.lax.dot_general(
        gw, b2, (((1,), (0,)), ((), ())),
        preferred_element_type=jnp.float32)
    return out, gw


def _moe_kernel(*refs):
    x_refs = refs[:NSX]
    wg_ref = refs[NSX]
    w1_refs = refs[NSX + 1:NSX + 1 + NSW]
    b1_ref, w2_ref, b2_ref, out_ref, gw_ref = refs[NSX + 1 + NSW:]
    wg = wg_ref[...]
    w1s = [r[...] for r in w1_refs]
    b1 = b1_ref[...]
    w2 = w2_ref[...]
    b2 = b2_ref[...]
    for t, x_ref in enumerate(x_refs):
        out_t, gw_t = _tile(x_ref[...], wg, w1s, b1, w2, b2)
        out_ref[t * TILE_B:(t + 1) * TILE_B, :] = out_t
        gw_ref[t * TILE_B:(t + 1) * TILE_B, :] = gw_t


@functools.partial(jax.jit, static_argnames=())
def kernel(x, Wg, W1, b1, W2, b2):
    B = x.shape[0]
    w1f = W1.reshape(HID, INPUT_DIM)
    w2f = W2.transpose(0, 2, 1).reshape(HID, EXPERT_DIM)
    b1f = b1.reshape(1, HID)
    full = lambda *shape: pl.BlockSpec(shape, lambda i: (0,) * len(shape))
    xspec = lambda t: pl.BlockSpec((TILE_B, INPUT_DIM),
                                   lambda i, t=t: (NSX * i + t, 0))
    wspec = lambda j: pl.BlockSpec((W1_ROWS, INPUT_DIM),
                                   lambda i, j=j: (j, 0))
    out, gw = pl.pallas_call(
        _moe_kernel,
        grid=(GRID,),
        in_specs=(
            [xspec(t) for t in range(NSX)]
            + [full(N_EXPERTS, INPUT_DIM)]
            + [wspec(j) for j in range(NSW)]
            + [full(1, HID), full(HID, EXPERT_DIM),
               full(N_EXPERTS, EXPERT_DIM)]
        ),
        out_specs=[
            pl.BlockSpec((NSX * TILE_B, EXPERT_DIM), lambda i: (i, 0)),
            pl.BlockSpec((NSX * TILE_B, N_EXPERTS), lambda i: (i, 0)),
        ],
        out_shape=[
            jax.ShapeDtypeStruct((B, EXPERT_DIM), jnp.float32),
            jax.ShapeDtypeStruct((B, N_EXPERTS), jnp.float32),
        ],
    )(*([x] * NSX + [Wg] + [w1f] * NSW + [b1f, w2f, b2]))
    return out, gw


# speedup vs baseline: 1.4395x; 1.4395x over previous
import functools

import jax
import jax.numpy as jnp
from jax.experimental import pallas as pl

INPUT_DIM = 768
N_EXPERTS = 8
EXPERT_DIM = 128
HID = N_EXPERTS * EXPERT_DIM
TOP_K = 2
GRID = 2
NSX = 4
NSW = 2
TILE_B = 2048 // (GRID * NSX)
W1_ROWS = HID // NSW


def _gating(logits):
    ids = jax.lax.broadcasted_iota(jnp.int32, logits.shape, 1)
    m1 = jnp.max(logits, axis=-1, keepdims=True)
    i1 = jnp.min(jnp.where(logits == m1, ids, N_EXPERTS),
                 axis=-1, keepdims=True)
    masked = jnp.where(ids == i1, -jnp.inf, logits)
    m2 = jnp.max(masked, axis=-1, keepdims=True)
    i2 = jnp.min(jnp.where(masked == m2, ids, N_EXPERTS),
                 axis=-1, keepdims=True)
    e2 = jnp.exp(m2 - m1)
    denom = 1.0 + e2
    return (jnp.where(ids == i1, 1.0 / denom, 0.0) +
            jnp.where(ids == i2, e2 / denom, 0.0))


def _tile(xt, wg, w1s, b1, w2, b2):
    logits = jax.lax.dot_general(
        xt, wg, (((1,), (1,)), ((), ())),
        preferred_element_type=jnp.float32)
    gw = _gating(logits)

    hparts = [jax.lax.dot_general(
        xt, w1, (((1,), (1,)), ((), ())),
        preferred_element_type=jnp.float32) for w1 in w1s]
    h = (hparts[0] if len(hparts) == 1
         else jnp.concatenate(hparts, axis=1)) + b1
    h = 0.5 * h * (1.0 + jax.lax.erf(h * 0.7071067811865476))

    hs = jnp.concatenate(
        [h[:, e * EXPERT_DIM:(e + 1) * EXPERT_DIM] * gw[:, e:e + 1]
         for e in range(N_EXPERTS)], axis=1)
    out = jax.lax.dot_general(
        hs, w2, (((1,), (0,)), ((), ())),
        preferred_element_type=jnp.float32)
    out = out + jax.lax.dot_general(
        gw, b2, (((1,), (0,)), ((), ())),
        preferred_element_type=jnp.float32)
    return out, gw


def _moe_kernel(*refs):
    x_refs = refs[:NSX]
    wg_ref = refs[NSX]
    w1_refs = refs[NSX + 1:NSX + 1 + NSW]
    b1_ref, w2_ref, b2_ref, out_ref, gw_ref = refs[NSX + 1 + NSW:]
    wg = wg_ref[...]
    w1s = [r[...] for r in w1_refs]
    b1 = b1_ref[...]
    w2 = w2_ref[...]
    b2 = b2_ref[...]
    for t, x_ref in enumerate(x_refs):
        out_t, gw_t = _tile(x_ref[...], wg, w1s, b1, w2, b2)
        out_ref[t * TILE_B:(t + 1) * TILE_B, :] = out_t
        gw_ref[t * TILE_B:(t + 1) * TILE_B, :] = gw_t


@functools.partial(jax.jit, static_argnames=())
def kernel(x, Wg, W1, b1, W2, b2):
    B = x.shape[0]
    w1f = W1.reshape(HID, INPUT_DIM)
    w2f = W2.transpose(0, 2, 1).reshape(HID, EXPERT_DIM)
    b1f = b1.reshape(1, HID)
    full = lambda *shape: pl.BlockSpec(shape, lambda i: (0,) * len(shape))
    xspec = lambda t: pl.BlockSpec((TILE_B, INPUT_DIM),
                                   lambda i, t=t: (NSX * i + t, 0))
    wspec = lambda j: pl.BlockSpec((W1_ROWS, INPUT_DIM),
                                   lambda i, j=j: (j, 0))
    out, gw = pl.pallas_call(
        _moe_kernel,
        grid=(GRID,),
        in_specs=(
            [xspec(t) for t in range(NSX)]
            + [full(N_EXPERTS, INPUT_DIM)]
            + [wspec(j) for j in range(NSW)]
            + [full(1, HID), full(HID, EXPERT_DIM),
               full(N_EXPERTS, EXPERT_DIM)]
        ),
        out_specs=[
            pl.BlockSpec((NSX * TILE_B, EXPERT_DIM), lambda i: (i, 0)),
            pl.BlockSpec((NSX * TILE_B, N_EXPERTS), lambda i: (i, 0)),
        ],
        out_shape=[
            jax.ShapeDtypeStruct((B, EXPERT_DIM), jnp.float32),
            jax.ShapeDtypeStruct((B, N_EXPERTS), jnp.float32),
        ],
    )(*([x] * NSX + [Wg] + [w1f] * NSW + [b1f, w2f, b2]))
    return out, gw
